# Initial kernel scaffold; baseline (speedup 1.0000x reference)
import jax
import jax.numpy as jnp
from jax.experimental import pallas as pl


def kernel(features, edge_index, W1, b1, W2, b2):
    raise NotImplementedError("write your pallas kernel here")


# speedup vs baseline: 9.8527x; 9.8527x over previous
import functools

import jax
import jax.numpy as jnp
from jax import lax
from jax.experimental import pallas as pl
from jax.experimental.pallas import tpu as pltpu
from jax.experimental.pallas import tpu_sc as plsc

N_NODES = 100000
N_EDGES = 1600000
F_HID = 16
F_OUT = 7
F_OUT_PAD = 8

ROW = 80
NROWS = N_EDGES // ROW
SB = 8
NBLK = NROWS // SB
NC = 2
NS = 16
NW = NC * NS
ZR = 1000
NZC = N_NODES // ZR
N_PAD = 100352
ZD = 1024
NZD = N_PAD // ZD

_mesh = plsc.VectorSubcoreMesh(core_axis_name="c", subcore_axis_name="s")
_sc_params = pltpu.CompilerParams(use_tc_tiling_on_sc=False)


@functools.partial(
    pl.kernel,
    out_type=(jax.ShapeDtypeStruct((N_PAD,), jnp.float32),
              jax.ShapeDtypeStruct((N_PAD,), jnp.float32)),
    mesh=_mesh,
    compiler_params=_sc_params,
    scratch_types=[
        pltpu.VMEM((SB, ROW), jnp.int32),
        pltpu.VMEM((ROW,), jnp.float32),
        pltpu.VMEM_SHARED((N_PAD,), jnp.float32),
    ],
)
def _deg_kernel(idx_hbm, zeros_hbm, dego_hbm, degi_hbm, ibuf, ones_v, acc):
    cid = lax.axis_index("c")
    sid = lax.axis_index("s")

    def _zero(k, carry):
        c = sid + k * NS

        @pl.when(c < NZD)
        def _():
            pltpu.sync_copy(zeros_hbm, acc.at[pl.ds(c * ZD, ZD)])

        return carry

    lax.fori_loop(0, (NZD + NS - 1) // NS, _zero, None)
    for k in range(ROW // 16):
        ones_v[pl.ds(k * 16, 16)] = jnp.ones((16,), jnp.float32)
    plsc.subcore_barrier()

    def _body(k, carry):
        b = sid + k * NS

        @pl.when(b < NBLK)
        def _():
            pltpu.sync_copy(idx_hbm.at[cid, pl.ds(b * SB, SB)], ibuf)
            for t in range(SB):
                pltpu.sync_copy(ones_v, acc.at[ibuf.at[t]], add=True)

        return carry

    lax.fori_loop(0, (NBLK + NS - 1) // NS, _body, None)
    plsc.subcore_barrier()

    def _write(k, carry):
        c = sid + k * NS

        @pl.when(c < NZD)
        def _():
            @pl.when(cid == 0)
            def _():
                pltpu.sync_copy(acc.at[pl.ds(c * ZD, ZD)],
                                dego_hbm.at[pl.ds(c * ZD, ZD)])

            @pl.when(cid == 1)
            def _():
                pltpu.sync_copy(acc.at[pl.ds(c * ZD, ZD)],
                                degi_hbm.at[pl.ds(c * ZD, ZD)])

        return carry

    lax.fori_loop(0, (NZD + NS - 1) // NS, _write, None)


def _make_agg(feat):
    @functools.partial(
        pl.kernel,
        out_type=jax.ShapeDtypeStruct((NC, N_NODES, feat), jnp.float32),
        mesh=_mesh,
        compiler_params=_sc_params,
        scratch_types=[
            pltpu.VMEM((SB, ROW), jnp.int32),
            pltpu.VMEM((SB, ROW), jnp.int32),
            pltpu.VMEM((ROW, feat), jnp.float32),
            pltpu.VMEM((ROW, feat), jnp.float32),
            pltpu.VMEM_SHARED((N_NODES, feat), jnp.float32),
            pltpu.SemaphoreType.DMA,
            pltpu.SemaphoreType.DMA,
        ],
    )
    def _agg(idx_hbm, h_hbm, zeros_hbm, out_hbm,
             sbuf, dbuf, msg0, msg1, acc, sem0, sem1):
        cid = lax.axis_index("c")
        sid = lax.axis_index("s")
        wid = sid * NC + cid

        def _zero(k, carry):
            c = sid + k * NS

            @pl.when(c < NZC)
            def _():
                pltpu.sync_copy(zeros_hbm, acc.at[pl.ds(c * ZR, ZR)])

            return carry

        lax.fori_loop(0, (NZC + NS - 1) // NS, _zero, None)
        plsc.subcore_barrier()

        msgs = (msg0, msg1)
        sems = (sem0, sem1)

        def _body(k, carry):
            b = wid + k * NW

            @pl.when(b < NBLK)
            def _():
                pltpu.sync_copy(idx_hbm.at[0, pl.ds(b * SB, SB)], sbuf)
                pltpu.sync_copy(idx_hbm.at[1, pl.ds(b * SB, SB)], dbuf)
                cp = pltpu.async_copy(h_hbm.at[sbuf.at[0]], msg0, sem0)
                for t in range(SB):
                    nxt = None
                    if t + 1 < SB:
                        nxt = pltpu.async_copy(h_hbm.at[sbuf.at[t + 1]],
                                               msgs[(t + 1) % 2],
                                               sems[(t + 1) % 2])
                    cp.wait()
                    pltpu.sync_copy(msgs[t % 2], acc.at[dbuf.at[t]], add=True)
                    cp = nxt

            return carry

        lax.fori_loop(0, (NBLK + NW - 1) // NW, _body, None)
        plsc.subcore_barrier()

        def _write(k, carry):
            c = sid + k * NS

            @pl.when(c < NZC)
            def _():
                pltpu.sync_copy(acc.at[pl.ds(c * ZR, ZR)],
                                out_hbm.at[cid, pl.ds(c * ZR, ZR)])

            return carry

        lax.fori_loop(0, (NZC + NS - 1) // NS, _write, None)

    return _agg


_agg16 = _make_agg(F_HID)
_agg8 = _make_agg(F_OUT_PAD)


def _mm_body(x_ref, w_ref, o_ref):
    o_ref[...] = jnp.dot(x_ref[...], w_ref[...],
                         preferred_element_type=jnp.float32)


def _matmul1(x, w):
    m, k = x.shape
    f = w.shape[1]
    bm = 2000
    return pl.pallas_call(
        _mm_body,
        grid=(m // bm,),
        in_specs=[pl.BlockSpec((bm, k), lambda i: (i, 0)),
                  pl.BlockSpec((k, f), lambda i: (0, 0))],
        out_specs=pl.BlockSpec((bm, f), lambda i: (i, 0)),
        out_shape=jax.ShapeDtypeStruct((m, f), jnp.float32),
    )(x, w)


def _scale_body(h_ref, d_ref, o_ref):
    o_ref[...] = h_ref[...] * lax.rsqrt(jnp.maximum(d_ref[...], 1.0))


def _scale(h, dego):
    bm = 4000
    return pl.pallas_call(
        _scale_body,
        grid=(N_NODES // bm,),
        in_specs=[pl.BlockSpec((bm, F_HID), lambda i: (i, 0)),
                  pl.BlockSpec((bm, 1), lambda i: (i, 0))],
        out_specs=pl.BlockSpec((bm, F_HID), lambda i: (i, 0)),
        out_shape=jax.ShapeDtypeStruct((N_NODES, F_HID), jnp.float32),
    )(h, dego)


def _mid_body(a_ref, di_ref, do_ref, b1_ref, w_ref, o_ref):
    ni = lax.rsqrt(jnp.maximum(di_ref[...], 1.0))
    x1 = jnp.maximum((a_ref[0] + a_ref[1]) * ni + b1_ref[...], 0.0)
    no = lax.rsqrt(jnp.maximum(do_ref[...], 1.0))
    o_ref[...] = jnp.dot(x1, w_ref[...],
                         preferred_element_type=jnp.float32) * no


def _mid(agg1, degi, dego, b1, w2p):
    bm = 4000
    return pl.pallas_call(
        _mid_body,
        grid=(N_NODES // bm,),
        in_specs=[pl.BlockSpec((NC, bm, F_HID), lambda i: (0, i, 0)),
                  pl.BlockSpec((bm, 1), lambda i: (i, 0)),
                  pl.BlockSpec((bm, 1), lambda i: (i, 0)),
                  pl.BlockSpec((F_HID,), lambda i: (0,)),
                  pl.BlockSpec((F_HID, F_OUT_PAD), lambda i: (0, 0))],
        out_specs=pl.BlockSpec((bm, F_OUT_PAD), lambda i: (i, 0)),
        out_shape=jax.ShapeDtypeStruct((N_NODES, F_OUT_PAD), jnp.float32),
    )(agg1, degi, dego, b1, w2p)


def _out_body(a_ref, di_ref, b2_ref, o_ref):
    ni = lax.rsqrt(jnp.maximum(di_ref[...], 1.0))
    o_ref[...] = (a_ref[0, :, :F_OUT] + a_ref[1, :, :F_OUT]) * ni + b2_ref[...]


def _outk(agg2, degi, b2):
    bm = 4000
    return pl.pallas_call(
        _out_body,
        grid=(N_NODES // bm,),
        in_specs=[pl.BlockSpec((NC, bm, F_OUT_PAD), lambda i: (0, i, 0)),
                  pl.BlockSpec((bm, 1), lambda i: (i, 0)),
                  pl.BlockSpec((F_OUT,), lambda i: (0,))],
        out_specs=pl.BlockSpec((bm, F_OUT), lambda i: (i, 0)),
        out_shape=jax.ShapeDtypeStruct((N_NODES, F_OUT), jnp.float32),
    )(agg2, degi, b2)


def kernel(features, edge_index, W1, b1, W2, b2):
    idx = edge_index.reshape(2, NROWS, ROW)
    dego, degi = _deg_kernel(idx, jnp.zeros((ZD,), jnp.float32))
    dego = dego[:N_NODES].reshape(N_NODES, 1)
    degi = degi[:N_NODES].reshape(N_NODES, 1)
    h1 = _matmul1(features, W1)
    h1s = _scale(h1, dego)
    agg1 = _agg16(idx, h1s, jnp.zeros((ZR, F_HID), jnp.float32))
    w2p = jnp.zeros((F_HID, F_OUT_PAD), jnp.float32).at[:, :F_OUT].set(W2)
    h2 = _mid(agg1, degi, dego, b1, w2p)
    agg2 = _agg8(idx, h2, jnp.zeros((ZR, F_OUT_PAD), jnp.float32))
    return _outk(agg2, degi, b2)


# speedup vs baseline: 12.1954x; 1.2378x over previous
import functools

import jax
import jax.numpy as jnp
from jax import lax
from jax.experimental import pallas as pl
from jax.experimental.pallas import tpu as pltpu
from jax.experimental.pallas import tpu_sc as plsc

N_NODES = 100000
N_EDGES = 1600000
F_HID = 16
F_OUT = 7
F_OUT_PAD = 8

ROW = 128
NROWS = N_EDGES // ROW
SB = 10
NBLK = NROWS // SB
NC = 2
NS = 16
NW = NC * NS
ZR = 1000
NZC = N_NODES // ZR
N_PAD = 100352
ZD = 1024
NZD = N_PAD // ZD

_mesh = plsc.VectorSubcoreMesh(core_axis_name="c", subcore_axis_name="s")
_sc_params = pltpu.CompilerParams(use_tc_tiling_on_sc=False)


@functools.partial(
    pl.kernel,
    out_type=(jax.ShapeDtypeStruct((N_PAD,), jnp.float32),
              jax.ShapeDtypeStruct((N_PAD,), jnp.float32)),
    mesh=_mesh,
    compiler_params=_sc_params,
    scratch_types=[
        pltpu.VMEM((SB, ROW), jnp.int32),
        pltpu.VMEM((ROW,), jnp.float32),
        pltpu.VMEM_SHARED((N_PAD,), jnp.float32),
        pltpu.SemaphoreType.DMA,
    ],
)
def _deg_kernel(idx_hbm, zeros_hbm, dego_hbm, degi_hbm, ibuf, ones_v, acc,
                ssem):
    cid = lax.axis_index("c")
    sid = lax.axis_index("s")

    def _zero(k, carry):
        c = sid + k * NS

        @pl.when(c < NZD)
        def _():
            pltpu.sync_copy(zeros_hbm, acc.at[pl.ds(c * ZD, ZD)])

        return carry

    lax.fori_loop(0, (NZD + NS - 1) // NS, _zero, None)
    for k in range(ROW // 16):
        ones_v[pl.ds(k * 16, 16)] = jnp.ones((16,), jnp.float32)
    plsc.subcore_barrier()

    def _body(k, carry):
        b = sid + k * NS

        @pl.when(b < NBLK)
        def _():
            pltpu.sync_copy(idx_hbm.at[cid, pl.ds(b * SB, SB)], ibuf)
            cps = [pltpu.async_copy(ones_v, acc.at[ibuf.at[t]], ssem,
                                    add=True)
                   for t in range(SB)]
            for cp in cps:
                cp.wait()

        return carry

    lax.fori_loop(0, (NBLK + NS - 1) // NS, _body, None)
    plsc.subcore_barrier()

    def _write(k, carry):
        c = sid + k * NS

        @pl.when(c < NZD)
        def _():
            @pl.when(cid == 0)
            def _():
                pltpu.sync_copy(acc.at[pl.ds(c * ZD, ZD)],
                                dego_hbm.at[pl.ds(c * ZD, ZD)])

            @pl.when(cid == 1)
            def _():
                pltpu.sync_copy(acc.at[pl.ds(c * ZD, ZD)],
                                degi_hbm.at[pl.ds(c * ZD, ZD)])

        return carry

    lax.fori_loop(0, (NZD + NS - 1) // NS, _write, None)


def _make_agg(feat):
    @functools.partial(
        pl.kernel,
        out_type=jax.ShapeDtypeStruct((NC, N_NODES, feat), jnp.float32),
        mesh=_mesh,
        compiler_params=_sc_params,
        scratch_types=[
            pltpu.VMEM((SB, ROW), jnp.int32),
            pltpu.VMEM((SB, ROW), jnp.int32),
            pltpu.VMEM((SB, ROW, feat), jnp.float32),
            pltpu.VMEM_SHARED((N_NODES, feat), jnp.float32),
            pltpu.SemaphoreType.DMA,
            pltpu.SemaphoreType.DMA,
        ],
    )
    def _agg(idx_hbm, h_hbm, zeros_hbm, out_hbm,
             sbuf, dbuf, msg, acc, gsem, ssem):
        cid = lax.axis_index("c")
        sid = lax.axis_index("s")
        wid = sid * NC + cid

        def _zero(k, carry):
            c = sid + k * NS

            @pl.when(c < NZC)
            def _():
                pltpu.sync_copy(zeros_hbm, acc.at[pl.ds(c * ZR, ZR)])

            return carry

        lax.fori_loop(0, (NZC + NS - 1) // NS, _zero, None)
        plsc.subcore_barrier()

        def _body(k, carry):
            b = wid + k * NW

            @pl.when(b < NBLK)
            def _():
                pltpu.sync_copy(idx_hbm.at[0, pl.ds(b * SB, SB)], sbuf)
                pltpu.sync_copy(idx_hbm.at[1, pl.ds(b * SB, SB)], dbuf)
                gcps = [pltpu.async_copy(h_hbm.at[sbuf.at[t]], msg.at[t], gsem)
                        for t in range(SB)]
                for cp in gcps:
                    cp.wait()
                scps = [pltpu.async_copy(msg.at[t], acc.at[dbuf.at[t]], ssem,
                                         add=True)
                        for t in range(SB)]
                for cp in scps:
                    cp.wait()

            return carry

        lax.fori_loop(0, (NBLK + NW - 1) // NW, _body, None)
        plsc.subcore_barrier()

        def _write(k, carry):
            c = sid + k * NS

            @pl.when(c < NZC)
            def _():
                pltpu.sync_copy(acc.at[pl.ds(c * ZR, ZR)],
                                out_hbm.at[cid, pl.ds(c * ZR, ZR)])

            return carry

        lax.fori_loop(0, (NZC + NS - 1) // NS, _write, None)

    return _agg


_agg16 = _make_agg(F_HID)
_agg8 = _make_agg(F_OUT_PAD)


def _mm_body(x_ref, w_ref, o_ref):
    o_ref[...] = jnp.dot(x_ref[...], w_ref[...],
                         preferred_element_type=jnp.float32)


def _matmul1(x, w):
    m, k = x.shape
    f = w.shape[1]
    bm = 2000
    return pl.pallas_call(
        _mm_body,
        grid=(m // bm,),
        in_specs=[pl.BlockSpec((bm, k), lambda i: (i, 0)),
                  pl.BlockSpec((k, f), lambda i: (0, 0))],
        out_specs=pl.BlockSpec((bm, f), lambda i: (i, 0)),
        out_shape=jax.ShapeDtypeStruct((m, f), jnp.float32),
    )(x, w)


def _scale_body(h_ref, d_ref, o_ref):
    o_ref[...] = h_ref[...] * lax.rsqrt(jnp.maximum(d_ref[...], 1.0))


def _scale(h, dego):
    bm = 4000
    return pl.pallas_call(
        _scale_body,
        grid=(N_NODES // bm,),
        in_specs=[pl.BlockSpec((bm, F_HID), lambda i: (i, 0)),
                  pl.BlockSpec((bm, 1), lambda i: (i, 0))],
        out_specs=pl.BlockSpec((bm, F_HID), lambda i: (i, 0)),
        out_shape=jax.ShapeDtypeStruct((N_NODES, F_HID), jnp.float32),
    )(h, dego)


def _mid_body(a_ref, di_ref, do_ref, b1_ref, w_ref, o_ref):
    ni = lax.rsqrt(jnp.maximum(di_ref[...], 1.0))
    x1 = jnp.maximum((a_ref[0] + a_ref[1]) * ni + b1_ref[...], 0.0)
    no = lax.rsqrt(jnp.maximum(do_ref[...], 1.0))
    o_ref[...] = jnp.dot(x1, w_ref[...],
                         preferred_element_type=jnp.float32) * no


def _mid(agg1, degi, dego, b1, w2p):
    bm = 4000
    return pl.pallas_call(
        _mid_body,
        grid=(N_NODES // bm,),
        in_specs=[pl.BlockSpec((NC, bm, F_HID), lambda i: (0, i, 0)),
                  pl.BlockSpec((bm, 1), lambda i: (i, 0)),
                  pl.BlockSpec((bm, 1), lambda i: (i, 0)),
                  pl.BlockSpec((F_HID,), lambda i: (0,)),
                  pl.BlockSpec((F_HID, F_OUT_PAD), lambda i: (0, 0))],
        out_specs=pl.BlockSpec((bm, F_OUT_PAD), lambda i: (i, 0)),
        out_shape=jax.ShapeDtypeStruct((N_NODES, F_OUT_PAD), jnp.float32),
    )(agg1, degi, dego, b1, w2p)


def _out_body(a_ref, di_ref, b2_ref, o_ref):
    ni = lax.rsqrt(jnp.maximum(di_ref[...], 1.0))
    o_ref[...] = (a_ref[0, :, :F_OUT] + a_ref[1, :, :F_OUT]) * ni + b2_ref[...]


def _outk(agg2, degi, b2):
    bm = 4000
    return pl.pallas_call(
        _out_body,
        grid=(N_NODES // bm,),
        in_specs=[pl.BlockSpec((NC, bm, F_OUT_PAD), lambda i: (0, i, 0)),
                  pl.BlockSpec((bm, 1), lambda i: (i, 0)),
                  pl.BlockSpec((F_OUT,), lambda i: (0,))],
        out_specs=pl.BlockSpec((bm, F_OUT), lambda i: (i, 0)),
        out_shape=jax.ShapeDtypeStruct((N_NODES, F_OUT), jnp.float32),
    )(agg2, degi, b2)


def kernel(features, edge_index, W1, b1, W2, b2):
    idx = edge_index.reshape(2, NROWS, ROW)
    dego, degi = _deg_kernel(idx, jnp.zeros((ZD,), jnp.float32))
    dego = dego[:N_NODES].reshape(N_NODES, 1)
    degi = degi[:N_NODES].reshape(N_NODES, 1)
    h1 = _matmul1(features, W1)
    h1s = _scale(h1, dego)
    agg1 = _agg16(idx, h1s, jnp.zeros((ZR, F_HID), jnp.float32))
    w2p = jnp.zeros((F_HID, F_OUT_PAD), jnp.float32).at[:, :F_OUT].set(W2)
    h2 = _mid(agg1, degi, dego, b1, w2p)
    agg2 = _agg8(idx, h2, jnp.zeros((ZR, F_OUT_PAD), jnp.float32))
    return _outk(agg2, degi, b2)


# speedup vs baseline: 12.4698x; 1.0225x over previous
import functools

import jax
import jax.numpy as jnp
from jax import lax
from jax.experimental import pallas as pl
from jax.experimental.pallas import tpu as pltpu
from jax.experimental.pallas import tpu_sc as plsc

N_NODES = 100000
N_EDGES = 1600000
F_HID = 16
F_OUT = 7
F_OUT_PAD = 8

ROW = 128
NROWS = N_EDGES // ROW
SB = 10
NBLK = NROWS // SB
NC = 2
NS = 16
NW = NC * NS
ZR = 1000
NZC = N_NODES // ZR
N_PAD = 100352
ZD = 1024
NZD = N_PAD // ZD

_mesh = plsc.VectorSubcoreMesh(core_axis_name="c", subcore_axis_name="s")
_sc_params = pltpu.CompilerParams(use_tc_tiling_on_sc=False)


@functools.partial(
    pl.kernel,
    out_type=(jax.ShapeDtypeStruct((N_PAD,), jnp.float32),
              jax.ShapeDtypeStruct((N_PAD,), jnp.float32)),
    mesh=_mesh,
    compiler_params=_sc_params,
    scratch_types=[
        pltpu.VMEM((SB, ROW), jnp.int32),
        pltpu.VMEM((ROW,), jnp.float32),
        pltpu.VMEM_SHARED((N_PAD,), jnp.float32),
        pltpu.SemaphoreType.DMA,
    ],
)
def _deg_kernel(idx_hbm, zeros_hbm, dego_hbm, degi_hbm, ibuf, ones_v, acc,
                ssem):
    cid = lax.axis_index("c")
    sid = lax.axis_index("s")

    def _zero(k, carry):
        c = sid + k * NS

        @pl.when(c < NZD)
        def _():
            pltpu.sync_copy(zeros_hbm, acc.at[pl.ds(c * ZD, ZD)])

        return carry

    lax.fori_loop(0, (NZD + NS - 1) // NS, _zero, None)
    for k in range(ROW // 16):
        ones_v[pl.ds(k * 16, 16)] = jnp.ones((16,), jnp.float32)
    plsc.subcore_barrier()

    def _body(k, carry):
        b = sid + k * NS

        @pl.when(b < NBLK)
        def _():
            pltpu.sync_copy(idx_hbm.at[cid, pl.ds(b * SB, SB)], ibuf)
            cps = [pltpu.async_copy(ones_v, acc.at[ibuf.at[t]], ssem,
                                    add=True)
                   for t in range(SB)]
            for cp in cps:
                cp.wait()

        return carry

    lax.fori_loop(0, (NBLK + NS - 1) // NS, _body, None)
    plsc.subcore_barrier()

    def _write(k, carry):
        c = sid + k * NS

        @pl.when(c < NZD)
        def _():
            @pl.when(cid == 0)
            def _():
                pltpu.sync_copy(acc.at[pl.ds(c * ZD, ZD)],
                                dego_hbm.at[pl.ds(c * ZD, ZD)])

            @pl.when(cid == 1)
            def _():
                pltpu.sync_copy(acc.at[pl.ds(c * ZD, ZD)],
                                degi_hbm.at[pl.ds(c * ZD, ZD)])

        return carry

    lax.fori_loop(0, (NZD + NS - 1) // NS, _write, None)


def _make_agg(feat):
    @functools.partial(
        pl.kernel,
        out_type=jax.ShapeDtypeStruct((NC, N_NODES, feat), jnp.float32),
        mesh=_mesh,
        compiler_params=_sc_params,
        scratch_types=[
            pltpu.VMEM((SB, ROW), jnp.int32),
            pltpu.VMEM((SB, ROW), jnp.int32),
            pltpu.VMEM((SB, ROW, feat), jnp.float32),
            pltpu.VMEM_SHARED((N_NODES, feat), jnp.float32),
            pltpu.SemaphoreType.DMA,
            pltpu.SemaphoreType.DMA,
        ],
    )
    def _agg(idx_hbm, h_hbm, zeros_hbm, out_hbm,
             sbuf, dbuf, msg, acc, gsem, ssem):
        cid = lax.axis_index("c")
        sid = lax.axis_index("s")
        wid = sid * NC + cid

        def _zero(k, carry):
            c = sid + k * NS

            @pl.when(c < NZC)
            def _():
                pltpu.sync_copy(zeros_hbm, acc.at[pl.ds(c * ZR, ZR)])

            return carry

        lax.fori_loop(0, (NZC + NS - 1) // NS, _zero, None)
        plsc.subcore_barrier()

        def _body(k, carry):
            b = wid + k * NW

            @pl.when(b < NBLK)
            def _():
                pltpu.sync_copy(idx_hbm.at[0, pl.ds(b * SB, SB)], sbuf)
                pltpu.sync_copy(idx_hbm.at[1, pl.ds(b * SB, SB)], dbuf)
                gcps = [pltpu.async_copy(h_hbm.at[sbuf.at[t]], msg.at[t], gsem)
                        for t in range(SB)]
                for cp in gcps:
                    cp.wait()
                scps = [pltpu.async_copy(msg.at[t], acc.at[dbuf.at[t]], ssem,
                                         add=True)
                        for t in range(SB)]
                for cp in scps:
                    cp.wait()

            return carry

        lax.fori_loop(0, (NBLK + NW - 1) // NW, _body, None)
        plsc.subcore_barrier()

        def _write(k, carry):
            c = sid + k * NS

            @pl.when(c < NZC)
            def _():
                pltpu.sync_copy(acc.at[pl.ds(c * ZR, ZR)],
                                out_hbm.at[cid, pl.ds(c * ZR, ZR)])

            return carry

        lax.fori_loop(0, (NZC + NS - 1) // NS, _write, None)

    return _agg


_agg16 = _make_agg(F_HID)
_agg8 = _make_agg(F_OUT_PAD)


def _mm_body(x_ref, w_ref, d_ref, o_ref):
    o_ref[...] = jnp.dot(x_ref[...], w_ref[...],
                         preferred_element_type=jnp.float32) * lax.rsqrt(
                             jnp.maximum(d_ref[...], 1.0))


def _matmul1(x, w, dego):
    m, k = x.shape
    f = w.shape[1]
    bm = 2000
    return pl.pallas_call(
        _mm_body,
        grid=(m // bm,),
        in_specs=[pl.BlockSpec((bm, k), lambda i: (i, 0)),
                  pl.BlockSpec((k, f), lambda i: (0, 0)),
                  pl.BlockSpec((bm, 1), lambda i: (i, 0))],
        out_specs=pl.BlockSpec((bm, f), lambda i: (i, 0)),
        out_shape=jax.ShapeDtypeStruct((m, f), jnp.float32),
    )(x, w, dego)


def _mid_body(a_ref, di_ref, do_ref, b1_ref, w_ref, o_ref):
    ni = lax.rsqrt(jnp.maximum(di_ref[...], 1.0))
    x1 = jnp.maximum((a_ref[0] + a_ref[1]) * ni + b1_ref[...], 0.0)
    no = lax.rsqrt(jnp.maximum(do_ref[...], 1.0))
    o_ref[...] = jnp.dot(x1, w_ref[...],
                         preferred_element_type=jnp.float32) * no


def _mid(agg1, degi, dego, b1, w2p):
    bm = 4000
    return pl.pallas_call(
        _mid_body,
        grid=(N_NODES // bm,),
        in_specs=[pl.BlockSpec((NC, bm, F_HID), lambda i: (0, i, 0)),
                  pl.BlockSpec((bm, 1), lambda i: (i, 0)),
                  pl.BlockSpec((bm, 1), lambda i: (i, 0)),
                  pl.BlockSpec((F_HID,), lambda i: (0,)),
                  pl.BlockSpec((F_HID, F_OUT_PAD), lambda i: (0, 0))],
        out_specs=pl.BlockSpec((bm, F_OUT_PAD), lambda i: (i, 0)),
        out_shape=jax.ShapeDtypeStruct((N_NODES, F_OUT_PAD), jnp.float32),
    )(agg1, degi, dego, b1, w2p)


def _out_body(a_ref, di_ref, b2_ref, o_ref):
    ni = lax.rsqrt(jnp.maximum(di_ref[...], 1.0))
    o_ref[...] = (a_ref[0, :, :F_OUT] + a_ref[1, :, :F_OUT]) * ni + b2_ref[...]


def _outk(agg2, degi, b2):
    bm = 4000
    return pl.pallas_call(
        _out_body,
        grid=(N_NODES // bm,),
        in_specs=[pl.BlockSpec((NC, bm, F_OUT_PAD), lambda i: (0, i, 0)),
                  pl.BlockSpec((bm, 1), lambda i: (i, 0)),
                  pl.BlockSpec((F_OUT,), lambda i: (0,))],
        out_specs=pl.BlockSpec((bm, F_OUT), lambda i: (i, 0)),
        out_shape=jax.ShapeDtypeStruct((N_NODES, F_OUT), jnp.float32),
    )(agg2, degi, b2)


def kernel(features, edge_index, W1, b1, W2, b2):
    idx = edge_index.reshape(2, NROWS, ROW)
    dego, degi = _deg_kernel(idx, jnp.zeros((ZD,), jnp.float32))
    dego = dego[:N_NODES].reshape(N_NODES, 1)
    degi = degi[:N_NODES].reshape(N_NODES, 1)
    h1s = _matmul1(features, W1, dego)
    agg1 = _agg16(idx, h1s, jnp.zeros((ZR, F_HID), jnp.float32))
    w2p = jnp.zeros((F_HID, F_OUT_PAD), jnp.float32).at[:, :F_OUT].set(W2)
    h2 = _mid(agg1, degi, dego, b1, w2p)
    agg2 = _agg8(idx, h2, jnp.zeros((ZR, F_OUT_PAD), jnp.float32))
    return _outk(agg2, degi, b2)


# speedup vs baseline: 19.1066x; 1.5322x over previous
import functools

import jax
import jax.numpy as jnp
from jax import lax
from jax.experimental import pallas as pl
from jax.experimental.pallas import tpu as pltpu
from jax.experimental.pallas import tpu_sc as plsc

N_NODES = 100000
N_EDGES = 1600000
F_HID = 16
F_OUT = 7
F_OUT_PAD = 8

ROW = 128
NROWS = N_EDGES // ROW
SB = 10
NBLK = NROWS // SB
NC = 2
NS = 16
NW = NC * NS
ZR = 1000
NZC = N_NODES // ZR
N_PAD = 100352
ZD = 1024
NZD = N_PAD // ZD

_mesh = plsc.VectorSubcoreMesh(core_axis_name="c", subcore_axis_name="s")
_sc_params = pltpu.CompilerParams(use_tc_tiling_on_sc=False)


@functools.partial(
    pl.kernel,
    out_type=(jax.ShapeDtypeStruct((N_PAD,), jnp.float32),
              jax.ShapeDtypeStruct((N_PAD,), jnp.float32)),
    mesh=_mesh,
    compiler_params=_sc_params,
    scratch_types=[
        pltpu.VMEM((SB, ROW), jnp.int32),
        pltpu.VMEM((ROW,), jnp.float32),
        pltpu.VMEM_SHARED((N_PAD,), jnp.float32),
        pltpu.SemaphoreType.DMA,
    ],
)
def _deg_kernel(idx_hbm, zeros_hbm, dego_hbm, degi_hbm, ibuf, ones_v, acc,
                ssem):
    cid = lax.axis_index("c")
    sid = lax.axis_index("s")

    def _zero(k, carry):
        c = sid + k * NS

        @pl.when(c < NZD)
        def _():
            pltpu.sync_copy(zeros_hbm, acc.at[pl.ds(c * ZD, ZD)])

        return carry

    lax.fori_loop(0, (NZD + NS - 1) // NS, _zero, None)
    for k in range(ROW // 16):
        ones_v[pl.ds(k * 16, 16)] = jnp.ones((16,), jnp.float32)
    plsc.subcore_barrier()

    def _body(k, carry):
        b = sid + k * NS

        @pl.when(b < NBLK)
        def _():
            pltpu.sync_copy(idx_hbm.at[cid, pl.ds(b * SB, SB)], ibuf)
            cps = [pltpu.async_copy(ones_v, acc.at[ibuf.at[t]], ssem,
                                    add=True)
                   for t in range(SB)]
            for cp in cps:
                cp.wait()

        return carry

    lax.fori_loop(0, (NBLK + NS - 1) // NS, _body, None)
    plsc.subcore_barrier()

    def _write(k, carry):
        c = sid + k * NS

        @pl.when(c < NZD)
        def _():
            @pl.when(cid == 0)
            def _():
                pltpu.sync_copy(acc.at[pl.ds(c * ZD, ZD)],
                                dego_hbm.at[pl.ds(c * ZD, ZD)])

            @pl.when(cid == 1)
            def _():
                pltpu.sync_copy(acc.at[pl.ds(c * ZD, ZD)],
                                degi_hbm.at[pl.ds(c * ZD, ZD)])

        return carry

    lax.fori_loop(0, (NZD + NS - 1) // NS, _write, None)


def _make_agg(feat):
    @functools.partial(
        pl.kernel,
        out_type=jax.ShapeDtypeStruct((NC, N_NODES, feat), jnp.float32),
        mesh=_mesh,
        compiler_params=_sc_params,
        scratch_types=[
            pltpu.VMEM((SB, ROW), jnp.int32),
            pltpu.VMEM((SB, ROW), jnp.int32),
            pltpu.VMEM((SB, ROW, feat), jnp.float32),
            pltpu.VMEM_SHARED((N_NODES, feat), jnp.float32),
            pltpu.SemaphoreType.DMA,
            pltpu.SemaphoreType.DMA,
        ],
    )
    def _agg(idx_hbm, h_hbm, zeros_hbm, out_hbm,
             sbuf, dbuf, msg, acc, gsem, ssem):
        cid = lax.axis_index("c")
        sid = lax.axis_index("s")
        wid = sid * NC + cid

        def _zero(k, carry):
            c = sid + k * NS

            @pl.when(c < NZC)
            def _():
                pltpu.sync_copy(zeros_hbm, acc.at[pl.ds(c * ZR, ZR)])

            return carry

        lax.fori_loop(0, (NZC + NS - 1) // NS, _zero, None)
        plsc.subcore_barrier()

        def _body(k, carry):
            b = wid + k * NW

            @pl.when(b < NBLK)
            def _():
                pltpu.sync_copy(idx_hbm.at[0, pl.ds(b * SB, SB)], sbuf)
                pltpu.sync_copy(idx_hbm.at[1, pl.ds(b * SB, SB)], dbuf)
                gcps = [pltpu.async_copy(h_hbm.at[sbuf.at[t]], msg.at[t], gsem)
                        for t in range(SB)]
                for cp in gcps:
                    cp.wait()
                scps = [pltpu.async_copy(msg.at[t], acc.at[dbuf.at[t]], ssem,
                                         add=True)
                        for t in range(SB)]
                for cp in scps:
                    cp.wait()

            return carry

        lax.fori_loop(0, (NBLK + NW - 1) // NW, _body, None)
        plsc.subcore_barrier()

        def _write(k, carry):
            c = sid + k * NS

            @pl.when(c < NZC)
            def _():
                pltpu.sync_copy(acc.at[pl.ds(c * ZR, ZR)],
                                out_hbm.at[cid, pl.ds(c * ZR, ZR)])

            return carry

        lax.fori_loop(0, (NZC + NS - 1) // NS, _write, None)

    return _agg


_agg16 = _make_agg(F_HID)
_agg8 = _make_agg(F_OUT_PAD)


def _mm_body(xt_ref, w_ref, d_ref, o_ref):
    h = lax.dot_general(xt_ref[...], w_ref[...], (((0,), (0,)), ((), ())),
                        preferred_element_type=jnp.float32)
    n = lax.rsqrt(jnp.maximum(d_ref[...], 1.0)).reshape(-1, 1)
    o_ref[...] = h * n


def _matmul1(xt, w, dego_pad):
    k, m = xt.shape
    f = w.shape[1]
    bm = 2048
    nb = (m + bm - 1) // bm
    return pl.pallas_call(
        _mm_body,
        grid=(nb,),
        in_specs=[pl.BlockSpec((k, bm), lambda i: (0, i)),
                  pl.BlockSpec((k, f), lambda i: (0, 0)),
                  pl.BlockSpec((bm,), lambda i: (i,))],
        out_specs=pl.BlockSpec((bm, f), lambda i: (i, 0)),
        out_shape=jax.ShapeDtypeStruct((m, f), jnp.float32),
    )(xt, w, dego_pad)


def _mid_body(a_ref, di_ref, do_ref, b1_ref, w_ref, o_ref):
    ni = lax.rsqrt(jnp.maximum(di_ref[...], 1.0)).reshape(-1, 1)
    x1 = jnp.maximum((a_ref[0] + a_ref[1]) * ni + b1_ref[...], 0.0)
    no = lax.rsqrt(jnp.maximum(do_ref[...], 1.0)).reshape(-1, 1)
    o_ref[...] = jnp.dot(x1, w_ref[...],
                         preferred_element_type=jnp.float32) * no


def _mid(agg1, degi, dego, b1, w2p):
    bm = 4096
    return pl.pallas_call(
        _mid_body,
        grid=((N_NODES + bm - 1) // bm,),
        in_specs=[pl.BlockSpec((NC, bm, F_HID), lambda i: (0, i, 0)),
                  pl.BlockSpec((bm,), lambda i: (i,)),
                  pl.BlockSpec((bm,), lambda i: (i,)),
                  pl.BlockSpec((F_HID,), lambda i: (0,)),
                  pl.BlockSpec((F_HID, F_OUT_PAD), lambda i: (0, 0))],
        out_specs=pl.BlockSpec((bm, F_OUT_PAD), lambda i: (i, 0)),
        out_shape=jax.ShapeDtypeStruct((N_NODES, F_OUT_PAD), jnp.float32),
    )(agg1, degi, dego, b1, w2p)


def _out_body(a_ref, di_ref, b2_ref, o_ref):
    ni = lax.rsqrt(jnp.maximum(di_ref[...], 1.0)).reshape(-1, 1)
    o_ref[...] = (a_ref[0, :, :F_OUT] + a_ref[1, :, :F_OUT]) * ni + b2_ref[...]


def _outk(agg2, degi, b2):
    bm = 4096
    return pl.pallas_call(
        _out_body,
        grid=((N_NODES + bm - 1) // bm,),
        in_specs=[pl.BlockSpec((NC, bm, F_OUT_PAD), lambda i: (0, i, 0)),
                  pl.BlockSpec((bm,), lambda i: (i,)),
                  pl.BlockSpec((F_OUT,), lambda i: (0,))],
        out_specs=pl.BlockSpec((bm, F_OUT), lambda i: (i, 0)),
        out_shape=jax.ShapeDtypeStruct((N_NODES, F_OUT), jnp.float32),
    )(agg2, degi, b2)


def kernel(features, edge_index, W1, b1, W2, b2):
    idx = edge_index.reshape(2, NROWS, ROW)
    dego, degi = _deg_kernel(idx, jnp.zeros((ZD,), jnp.float32))
    h1s = _matmul1(features.T, W1, dego)
    agg1 = _agg16(idx, h1s, jnp.zeros((ZR, F_HID), jnp.float32))
    w2p = jnp.zeros((F_HID, F_OUT_PAD), jnp.float32).at[:, :F_OUT].set(W2)
    h2 = _mid(agg1, degi, dego, b1, w2p)
    agg2 = _agg8(idx, h2, jnp.zeros((ZR, F_OUT_PAD), jnp.float32))
    return _outk(agg2, degi, b2)


# speedup vs baseline: 22.2212x; 1.1630x over previous
import functools

import jax
import jax.numpy as jnp
from jax import lax
from jax.experimental import pallas as pl
from jax.experimental.pallas import tpu as pltpu
from jax.experimental.pallas import tpu_sc as plsc

N_NODES = 100000
N_EDGES = 1600000
F_HID = 16
F_OUT = 7
F_OUT_PAD = 8

ROW = 128
NROWS = N_EDGES // ROW
SB = 10
NBLK = NROWS // SB
NC = 2
NS = 16
NW = NC * NS
ZR = 1000
NZC = N_NODES // ZR
N_PAD = 100352
ZD = 1024
NZD = N_PAD // ZD

_mesh = plsc.VectorSubcoreMesh(core_axis_name="c", subcore_axis_name="s")
_sc_params = pltpu.CompilerParams(use_tc_tiling_on_sc=False)


@functools.partial(
    pl.kernel,
    out_type=(jax.ShapeDtypeStruct((N_PAD,), jnp.float32),
              jax.ShapeDtypeStruct((N_PAD,), jnp.float32)),
    mesh=_mesh,
    compiler_params=_sc_params,
    scratch_types=[
        pltpu.VMEM((SB, ROW), jnp.int32),
        pltpu.VMEM((ROW,), jnp.float32),
        pltpu.VMEM_SHARED((N_PAD,), jnp.float32),
        pltpu.SemaphoreType.DMA,
    ],
)
def _deg_kernel(idx_hbm, zeros_hbm, dego_hbm, degi_hbm, ibuf, ones_v, acc,
                ssem):
    cid = lax.axis_index("c")
    sid = lax.axis_index("s")

    def _zero(k, carry):
        c = sid + k * NS

        @pl.when(c < NZD)
        def _():
            pltpu.sync_copy(zeros_hbm, acc.at[pl.ds(c * ZD, ZD)])

        return carry

    lax.fori_loop(0, (NZD + NS - 1) // NS, _zero, None)
    for k in range(ROW // 16):
        ones_v[pl.ds(k * 16, 16)] = jnp.ones((16,), jnp.float32)
    plsc.subcore_barrier()

    def _body(k, carry):
        b = sid + k * NS

        @pl.when(b < NBLK)
        def _():
            pltpu.sync_copy(idx_hbm.at[cid, pl.ds(b * SB, SB)], ibuf)
            cps = [pltpu.async_copy(ones_v, acc.at[ibuf.at[t]], ssem,
                                    add=True)
                   for t in range(SB)]
            for cp in cps:
                cp.wait()

        return carry

    lax.fori_loop(0, (NBLK + NS - 1) // NS, _body, None)
    plsc.subcore_barrier()

    def _write(k, carry):
        c = sid + k * NS

        @pl.when(c < NZD)
        def _():
            @pl.when(cid == 0)
            def _():
                pltpu.sync_copy(acc.at[pl.ds(c * ZD, ZD)],
                                dego_hbm.at[pl.ds(c * ZD, ZD)])

            @pl.when(cid == 1)
            def _():
                pltpu.sync_copy(acc.at[pl.ds(c * ZD, ZD)],
                                degi_hbm.at[pl.ds(c * ZD, ZD)])

        return carry

    lax.fori_loop(0, (NZD + NS - 1) // NS, _write, None)


def _make_agg(feat):
    @functools.partial(
        pl.kernel,
        out_type=jax.ShapeDtypeStruct((NC, N_NODES, feat), jnp.float32),
        mesh=_mesh,
        compiler_params=_sc_params,
        scratch_types=[
            pltpu.VMEM((SB, ROW), jnp.int32),
            pltpu.VMEM((SB, ROW), jnp.int32),
            pltpu.VMEM((SB, ROW, feat), jnp.float32),
            pltpu.VMEM_SHARED((N_NODES, feat), jnp.float32),
            pltpu.SemaphoreType.DMA,
            pltpu.SemaphoreType.DMA,
        ],
    )
    def _agg(idx_hbm, h_hbm, zeros_hbm, out_hbm,
             sbuf, dbuf, msg, acc, gsem, ssem):
        cid = lax.axis_index("c")
        sid = lax.axis_index("s")
        wid = sid * NC + cid

        def _zero(k, carry):
            c = sid + k * NS

            @pl.when(c < NZC)
            def _():
                pltpu.sync_copy(zeros_hbm, acc.at[pl.ds(c * ZR, ZR)])

            return carry

        lax.fori_loop(0, (NZC + NS - 1) // NS, _zero, None)
        plsc.subcore_barrier()

        def _body(k, carry):
            b = wid + k * NW

            @pl.when(b < NBLK)
            def _():
                pltpu.sync_copy(idx_hbm.at[0, pl.ds(b * SB, SB)], sbuf)
                pltpu.sync_copy(idx_hbm.at[1, pl.ds(b * SB, SB)], dbuf)
                gcps = [pltpu.async_copy(h_hbm.at[sbuf.at[t]], msg.at[t], gsem)
                        for t in range(SB)]
                for cp in gcps:
                    cp.wait()
                scps = [pltpu.async_copy(msg.at[t], acc.at[dbuf.at[t]], ssem,
                                         add=True)
                        for t in range(SB)]
                for cp in scps:
                    cp.wait()

            return carry

        lax.fori_loop(0, (NBLK + NW - 1) // NW, _body, None)
        plsc.subcore_barrier()

        def _write(k, carry):
            c = sid + k * NS

            @pl.when(c < NZC)
            def _():
                pltpu.sync_copy(acc.at[pl.ds(c * ZR, ZR)],
                                out_hbm.at[cid, pl.ds(c * ZR, ZR)])

            return carry

        lax.fori_loop(0, (NZC + NS - 1) // NS, _write, None)

    return _agg


_agg16 = _make_agg(F_HID)
NP = N_NODES // 8


def _mm_body(xt_ref, w_ref, d_ref, o_ref):
    h = lax.dot_general(xt_ref[...], w_ref[...], (((0,), (0,)), ((), ())),
                        preferred_element_type=jnp.float32)
    n = lax.rsqrt(jnp.maximum(d_ref[...], 1.0)).reshape(-1, 1)
    o_ref[...] = h * n


def _matmul1(xt, w, dego_pad):
    k, m = xt.shape
    f = w.shape[1]
    bm = 2048
    nb = (m + bm - 1) // bm
    return pl.pallas_call(
        _mm_body,
        grid=(nb,),
        in_specs=[pl.BlockSpec((k, bm), lambda i: (0, i)),
                  pl.BlockSpec((k, f), lambda i: (0, 0)),
                  pl.BlockSpec((bm,), lambda i: (i,))],
        out_specs=pl.BlockSpec((bm, f), lambda i: (i, 0)),
        out_shape=jax.ShapeDtypeStruct((m, f), jnp.float32),
    )(xt, w, dego_pad)


def _packed_norm(d8, e):
    return jnp.dot(lax.rsqrt(jnp.maximum(d8, 1.0)), e,
                   preferred_element_type=jnp.float32)


def _mid_body(a_ref, di_ref, do_ref, e_ref, b1t_ref, bd_ref, o_ref):
    ni = _packed_norm(di_ref[...], e_ref[...])
    x1 = jnp.maximum((a_ref[0] + a_ref[1]) * ni + b1t_ref[...], 0.0)
    no = _packed_norm(do_ref[...], e_ref[...])
    o_ref[...] = jnp.dot(x1, bd_ref[...],
                         preferred_element_type=jnp.float32) * no


def _mid(agg1p, degi8, dego8, e, b1t, bd):
    bp = 512
    return pl.pallas_call(
        _mid_body,
        grid=((NP + bp - 1) // bp,),
        in_specs=[pl.BlockSpec((NC, bp, 128), lambda i: (0, i, 0)),
                  pl.BlockSpec((bp, 8), lambda i: (i, 0)),
                  pl.BlockSpec((bp, 8), lambda i: (i, 0)),
                  pl.BlockSpec((8, 128), lambda i: (0, 0)),
                  pl.BlockSpec((128,), lambda i: (0,)),
                  pl.BlockSpec((128, 128), lambda i: (0, 0))],
        out_specs=pl.BlockSpec((bp, 128), lambda i: (i, 0)),
        out_shape=jax.ShapeDtypeStruct((NP, 128), jnp.float32),
    )(agg1p, degi8, dego8, e, b1t, bd)


def _out_body(a_ref, di_ref, e_ref, b2t_ref, o_ref):
    ni = _packed_norm(di_ref[...], e_ref[...])
    o_ref[...] = (a_ref[0] + a_ref[1]) * ni + b2t_ref[...]


def _outk(agg2p, degi8, e, b2t):
    bp = 512
    return pl.pallas_call(
        _out_body,
        grid=((NP + bp - 1) // bp,),
        in_specs=[pl.BlockSpec((NC, bp, 128), lambda i: (0, i, 0)),
                  pl.BlockSpec((bp, 8), lambda i: (i, 0)),
                  pl.BlockSpec((8, 128), lambda i: (0, 0)),
                  pl.BlockSpec((128,), lambda i: (0,))],
        out_specs=pl.BlockSpec((bp, 128), lambda i: (i, 0)),
        out_shape=jax.ShapeDtypeStruct((NP, 128), jnp.float32),
    )(agg2p, degi8, e, b2t)


def kernel(features, edge_index, W1, b1, W2, b2):
    idx = edge_index.reshape(2, NROWS, ROW)
    dego, degi = _deg_kernel(idx, jnp.zeros((ZD,), jnp.float32))
    zeros16 = jnp.zeros((ZR, F_HID), jnp.float32)
    h1s = _matmul1(features.T, W1, dego)
    agg1 = _agg16(idx, h1s, zeros16)
    degi8 = degi.reshape(N_PAD // 8, 8)
    dego8 = dego.reshape(N_PAD // 8, 8)
    e = jnp.kron(jnp.eye(8, dtype=jnp.float32),
                 jnp.ones((1, 16), jnp.float32))
    w2p = jnp.zeros((F_HID, 16), jnp.float32).at[:, :F_OUT].set(W2)
    bd = jnp.kron(jnp.eye(8, dtype=jnp.float32), w2p)
    b1t = jnp.tile(b1, 8)
    b2t = jnp.tile(jnp.zeros((16,), jnp.float32).at[:F_OUT].set(b2), 8)
    h2p = _mid(agg1.reshape(NC, NP, 128), degi8, dego8, e, b1t, bd)
    agg2 = _agg16(idx, h2p.reshape(N_NODES, F_HID), zeros16)
    outp = _outk(agg2.reshape(NC, NP, 128), degi8, e, b2t)
    return outp.reshape(N_NODES, F_HID)[:, :F_OUT]


# speedup vs baseline: 22.8477x; 1.0282x over previous
import functools

import jax
import jax.numpy as jnp
from jax import lax
from jax.experimental import pallas as pl
from jax.experimental.pallas import tpu as pltpu
from jax.experimental.pallas import tpu_sc as plsc

N_NODES = 100000
N_EDGES = 1600000
F_HID = 16
F_OUT = 7
F_OUT_PAD = 8

ROW = 128
NROWS = N_EDGES // ROW
SB = 5
NBLK = NROWS // SB
NC = 2
NS = 16
NW = NC * NS
ZR = 1000
NZC = N_NODES // ZR
N_PAD = 100352
ZD = 1024
NZD = N_PAD // ZD

_mesh = plsc.VectorSubcoreMesh(core_axis_name="c", subcore_axis_name="s")
_sc_params = pltpu.CompilerParams(use_tc_tiling_on_sc=False)


@functools.partial(
    pl.kernel,
    out_type=(jax.ShapeDtypeStruct((N_PAD,), jnp.float32),
              jax.ShapeDtypeStruct((N_PAD,), jnp.float32)),
    mesh=_mesh,
    compiler_params=_sc_params,
    scratch_types=[
        pltpu.VMEM((SB, ROW), jnp.int32),
        pltpu.VMEM((SB, ROW), jnp.int32),
        pltpu.VMEM((ROW,), jnp.float32),
        pltpu.VMEM_SHARED((N_PAD,), jnp.float32),
        pltpu.SemaphoreType.DMA,
        pltpu.SemaphoreType.DMA,
    ],
)
def _deg_kernel(idx_hbm, zeros_hbm, dego_hbm, degi_hbm, ibufa, ibufb, ones_v,
                acc, ssema, ssemb):
    cid = lax.axis_index("c")
    sid = lax.axis_index("s")

    def _zero(k, carry):
        c = sid + k * NS

        @pl.when(c < NZD)
        def _():
            pltpu.sync_copy(zeros_hbm, acc.at[pl.ds(c * ZD, ZD)])

        return carry

    lax.fori_loop(0, (NZD + NS - 1) // NS, _zero, None)
    for k in range(ROW // 16):
        ones_v[pl.ds(k * 16, 16)] = jnp.ones((16,), jnp.float32)
    plsc.subcore_barrier()

    def _body(k, carry):
        for half, (ibuf, ssem) in enumerate(((ibufa, ssema), (ibufb, ssemb))):
            b = sid + (2 * k + half) * NS

            @pl.when(b < NBLK)
            def _():
                @pl.when(k > 0)
                def _():
                    for t in range(SB):
                        pltpu.make_async_copy(
                            ones_v, acc.at[ibuf.at[t]], ssem).wait()

                pltpu.sync_copy(idx_hbm.at[cid, pl.ds(b * SB, SB)], ibuf)
                for t in range(SB):
                    pltpu.async_copy(ones_v, acc.at[ibuf.at[t]], ssem,
                                     add=True)

        return carry

    lax.fori_loop(0, (NBLK + 2 * NS - 1) // (2 * NS), _body, None)
    for ibuf, ssem in ((ibufa, ssema), (ibufb, ssemb)):
        for t in range(SB):
            pltpu.make_async_copy(ones_v, acc.at[ibuf.at[t]], ssem).wait()
    plsc.subcore_barrier()

    def _write(k, carry):
        c = sid + k * NS

        @pl.when(c < NZD)
        def _():
            @pl.when(cid == 0)
            def _():
                pltpu.sync_copy(acc.at[pl.ds(c * ZD, ZD)],
                                dego_hbm.at[pl.ds(c * ZD, ZD)])

            @pl.when(cid == 1)
            def _():
                pltpu.sync_copy(acc.at[pl.ds(c * ZD, ZD)],
                                degi_hbm.at[pl.ds(c * ZD, ZD)])

        return carry

    lax.fori_loop(0, (NZD + NS - 1) // NS, _write, None)


def _make_agg(feat):
    @functools.partial(
        pl.kernel,
        out_type=jax.ShapeDtypeStruct((NC, N_NODES, feat), jnp.float32),
        mesh=_mesh,
        compiler_params=_sc_params,
        scratch_types=[
            pltpu.VMEM((SB, ROW), jnp.int32),
            pltpu.VMEM((SB, ROW), jnp.int32),
            pltpu.VMEM((SB, ROW), jnp.int32),
            pltpu.VMEM((SB, ROW), jnp.int32),
            pltpu.VMEM((SB, ROW, feat), jnp.float32),
            pltpu.VMEM((SB, ROW, feat), jnp.float32),
            pltpu.VMEM_SHARED((N_NODES, feat), jnp.float32),
            pltpu.SemaphoreType.DMA,
            pltpu.SemaphoreType.DMA,
            pltpu.SemaphoreType.DMA,
            pltpu.SemaphoreType.DMA,
        ],
    )
    def _agg(idx_hbm, h_hbm, zeros_hbm, out_hbm,
             sbufa, dbufa, sbufb, dbufb, msga, msgb, acc,
             gsema, gsemb, ssema, ssemb):
        cid = lax.axis_index("c")
        sid = lax.axis_index("s")
        wid = sid * NC + cid

        def _zero(k, carry):
            c = sid + k * NS

            @pl.when(c < NZC)
            def _():
                pltpu.sync_copy(zeros_hbm, acc.at[pl.ds(c * ZR, ZR)])

            return carry

        lax.fori_loop(0, (NZC + NS - 1) // NS, _zero, None)
        plsc.subcore_barrier()

        sets = ((sbufa, dbufa, msga, gsema, ssema),
                (sbufb, dbufb, msgb, gsemb, ssemb))

        def _body(k, carry):
            for half, (sbuf, dbuf, msg, gsem, ssem) in enumerate(sets):
                b = wid + (2 * k + half) * NW

                @pl.when(b < NBLK)
                def _():
                    @pl.when(k > 0)
                    def _():
                        for t in range(SB):
                            pltpu.make_async_copy(
                                msg.at[t], acc.at[dbuf.at[t]], ssem).wait()

                    pltpu.sync_copy(idx_hbm.at[0, pl.ds(b * SB, SB)], sbuf)
                    pltpu.sync_copy(idx_hbm.at[1, pl.ds(b * SB, SB)], dbuf)
                    for t in range(SB):
                        pltpu.async_copy(h_hbm.at[sbuf.at[t]], msg.at[t], gsem)

            for half, (sbuf, dbuf, msg, gsem, ssem) in enumerate(sets):
                b = wid + (2 * k + half) * NW

                @pl.when(b < NBLK)
                def _():
                    for t in range(SB):
                        pltpu.make_async_copy(
                            h_hbm.at[sbuf.at[t]], msg.at[t], gsem).wait()
                    for t in range(SB):
                        pltpu.async_copy(msg.at[t], acc.at[dbuf.at[t]], ssem,
                                         add=True)

            return carry

        lax.fori_loop(0, (NBLK + 2 * NW - 1) // (2 * NW), _body, None)
        for sbuf, dbuf, msg, gsem, ssem in sets:
            for t in range(SB):
                pltpu.make_async_copy(
                    msg.at[t], acc.at[dbuf.at[t]], ssem).wait()
        plsc.subcore_barrier()

        def _write(k, carry):
            c = sid + k * NS

            @pl.when(c < NZC)
            def _():
                pltpu.sync_copy(acc.at[pl.ds(c * ZR, ZR)],
                                out_hbm.at[cid, pl.ds(c * ZR, ZR)])

            return carry

        lax.fori_loop(0, (NZC + NS - 1) // NS, _write, None)

    return _agg


_agg16 = _make_agg(F_HID)
NP = N_NODES // 8


def _mm_body(xt_ref, w_ref, d_ref, o_ref):
    h = lax.dot_general(xt_ref[...], w_ref[...], (((0,), (0,)), ((), ())),
                        preferred_element_type=jnp.float32)
    n = lax.rsqrt(jnp.maximum(d_ref[...], 1.0)).reshape(-1, 1)
    o_ref[...] = h * n


def _matmul1(xt, w, dego_pad):
    k, m = xt.shape
    f = w.shape[1]
    bm = 2048
    nb = (m + bm - 1) // bm
    return pl.pallas_call(
        _mm_body,
        grid=(nb,),
        in_specs=[pl.BlockSpec((k, bm), lambda i: (0, i)),
                  pl.BlockSpec((k, f), lambda i: (0, 0)),
                  pl.BlockSpec((bm,), lambda i: (i,))],
        out_specs=pl.BlockSpec((bm, f), lambda i: (i, 0)),
        out_shape=jax.ShapeDtypeStruct((m, f), jnp.float32),
    )(xt, w, dego_pad)


def _packed_norm(d8, e):
    return jnp.dot(lax.rsqrt(jnp.maximum(d8, 1.0)), e,
                   preferred_element_type=jnp.float32)


def _mid_body(a_ref, di_ref, do_ref, e_ref, b1t_ref, bd_ref, o_ref):
    ni = _packed_norm(di_ref[...], e_ref[...])
    x1 = jnp.maximum((a_ref[0] + a_ref[1]) * ni + b1t_ref[...], 0.0)
    no = _packed_norm(do_ref[...], e_ref[...])
    o_ref[...] = jnp.dot(x1, bd_ref[...],
                         preferred_element_type=jnp.float32) * no


def _mid(agg1p, degi8, dego8, e, b1t, bd):
    bp = 512
    return pl.pallas_call(
        _mid_body,
        grid=((NP + bp - 1) // bp,),
        in_specs=[pl.BlockSpec((NC, bp, 128), lambda i: (0, i, 0)),
                  pl.BlockSpec((bp, 8), lambda i: (i, 0)),
                  pl.BlockSpec((bp, 8), lambda i: (i, 0)),
                  pl.BlockSpec((8, 128), lambda i: (0, 0)),
                  pl.BlockSpec((128,), lambda i: (0,)),
                  pl.BlockSpec((128, 128), lambda i: (0, 0))],
        out_specs=pl.BlockSpec((bp, 128), lambda i: (i, 0)),
        out_shape=jax.ShapeDtypeStruct((NP, 128), jnp.float32),
    )(agg1p, degi8, dego8, e, b1t, bd)


def _out_body(a_ref, di_ref, e_ref, b2t_ref, o_ref):
    ni = _packed_norm(di_ref[...], e_ref[...])
    o_ref[...] = (a_ref[0] + a_ref[1]) * ni + b2t_ref[...]


def _outk(agg2p, degi8, e, b2t):
    bp = 512
    return pl.pallas_call(
        _out_body,
        grid=((NP + bp - 1) // bp,),
        in_specs=[pl.BlockSpec((NC, bp, 128), lambda i: (0, i, 0)),
                  pl.BlockSpec((bp, 8), lambda i: (i, 0)),
                  pl.BlockSpec((8, 128), lambda i: (0, 0)),
                  pl.BlockSpec((128,), lambda i: (0,))],
        out_specs=pl.BlockSpec((bp, 128), lambda i: (i, 0)),
        out_shape=jax.ShapeDtypeStruct((NP, 128), jnp.float32),
    )(agg2p, degi8, e, b2t)


def kernel(features, edge_index, W1, b1, W2, b2):
    idx = edge_index.reshape(2, NROWS, ROW)
    dego, degi = _deg_kernel(idx, jnp.zeros((ZD,), jnp.float32))
    zeros16 = jnp.zeros((ZR, F_HID), jnp.float32)
    h1s = _matmul1(features.T, W1, dego)
    agg1 = _agg16(idx, h1s, zeros16)
    degi8 = degi.reshape(N_PAD // 8, 8)
    dego8 = dego.reshape(N_PAD // 8, 8)
    e = jnp.kron(jnp.eye(8, dtype=jnp.float32),
                 jnp.ones((1, 16), jnp.float32))
    w2p = jnp.zeros((F_HID, 16), jnp.float32).at[:, :F_OUT].set(W2)
    bd = jnp.kron(jnp.eye(8, dtype=jnp.float32), w2p)
    b1t = jnp.tile(b1, 8)
    b2t = jnp.tile(jnp.zeros((16,), jnp.float32).at[:F_OUT].set(b2), 8)
    h2p = _mid(agg1.reshape(NC, NP, 128), degi8, dego8, e, b1t, bd)
    agg2 = _agg16(idx, h2p.reshape(N_NODES, F_HID), zeros16)
    outp = _outk(agg2.reshape(NC, NP, 128), degi8, e, b2t)
    return outp.reshape(N_NODES, F_HID)[:, :F_OUT]


# speedup vs baseline: 25.7130x; 1.1254x over previous
import functools

import jax
import jax.numpy as jnp
from jax import lax
from jax.experimental import pallas as pl
from jax.experimental.pallas import tpu as pltpu
from jax.experimental.pallas import tpu_sc as plsc

N_NODES = 100000
N_EDGES = 1600000
F_HID = 16
F_OUT = 7
F_OUT_PAD = 8

ROW = 128
NROWS = N_EDGES // ROW
SB = 5
NBLK = NROWS // SB
NC = 2
NS = 16
NW = NC * NS
ZR = 1000
NZC = N_NODES // ZR
N_PAD = 100352
ZD = 1024
NZD = N_PAD // ZD

_mesh = plsc.VectorSubcoreMesh(core_axis_name="c", subcore_axis_name="s")
_sc_params = pltpu.CompilerParams(use_tc_tiling_on_sc=False)


@functools.partial(
    pl.kernel,
    out_type=(jax.ShapeDtypeStruct((N_PAD,), jnp.float32),
              jax.ShapeDtypeStruct((N_PAD,), jnp.float32)),
    mesh=_mesh,
    compiler_params=_sc_params,
    scratch_types=[
        pltpu.VMEM((SB, 2, ROW), jnp.int32),
        pltpu.VMEM((SB, 2, ROW), jnp.int32),
        pltpu.VMEM((ROW,), jnp.float32),
        pltpu.VMEM_SHARED((N_PAD,), jnp.float32),
        pltpu.SemaphoreType.DMA,
        pltpu.SemaphoreType.DMA,
    ],
)
def _deg_kernel(idx_hbm, zeros_hbm, dego_hbm, degi_hbm, ibufa, ibufb, ones_v,
                acc, ssema, ssemb):
    cid = lax.axis_index("c")
    sid = lax.axis_index("s")

    def _zero(k, carry):
        c = sid + k * NS

        @pl.when(c < NZD)
        def _():
            pltpu.sync_copy(zeros_hbm, acc.at[pl.ds(c * ZD, ZD)])

        return carry

    lax.fori_loop(0, (NZD + NS - 1) // NS, _zero, None)
    for k in range(ROW // 16):
        ones_v[pl.ds(k * 16, 16)] = jnp.ones((16,), jnp.float32)
    plsc.subcore_barrier()

    def _body(k, carry):
        for half, (ibuf, ssem) in enumerate(((ibufa, ssema), (ibufb, ssemb))):
            b = sid + (2 * k + half) * NS

            @pl.when(b < NBLK)
            def _():
                @pl.when(k > 0)
                def _():
                    for t in range(SB):
                        pltpu.make_async_copy(
                            ones_v, acc.at[ibuf.at[t, cid]], ssem).wait()

                pltpu.sync_copy(idx_hbm.at[pl.ds(b * SB, SB)], ibuf)
                for t in range(SB):
                    pltpu.async_copy(ones_v, acc.at[ibuf.at[t, cid]], ssem,
                                     add=True)

        return carry

    lax.fori_loop(0, (NBLK + 2 * NS - 1) // (2 * NS), _body, None)
    for ibuf, ssem in ((ibufa, ssema), (ibufb, ssemb)):
        for t in range(SB):
            pltpu.make_async_copy(ones_v, acc.at[ibuf.at[t, cid]],
                                  ssem).wait()
    plsc.subcore_barrier()

    def _write(k, carry):
        c = sid + k * NS

        @pl.when(c < NZD)
        def _():
            @pl.when(cid == 0)
            def _():
                pltpu.sync_copy(acc.at[pl.ds(c * ZD, ZD)],
                                dego_hbm.at[pl.ds(c * ZD, ZD)])

            @pl.when(cid == 1)
            def _():
                pltpu.sync_copy(acc.at[pl.ds(c * ZD, ZD)],
                                degi_hbm.at[pl.ds(c * ZD, ZD)])

        return carry

    lax.fori_loop(0, (NZD + NS - 1) // NS, _write, None)


def _make_agg(feat):
    @functools.partial(
        pl.kernel,
        out_type=jax.ShapeDtypeStruct((NC, N_NODES, feat), jnp.float32),
        mesh=_mesh,
        compiler_params=_sc_params,
        scratch_types=[
            pltpu.VMEM((SB, 2, ROW), jnp.int32),
            pltpu.VMEM((SB, 2, ROW), jnp.int32),
            pltpu.VMEM((SB, ROW, feat), jnp.float32),
            pltpu.VMEM((SB, ROW, feat), jnp.float32),
            pltpu.VMEM_SHARED((N_NODES, feat), jnp.float32),
            pltpu.SemaphoreType.DMA,
            pltpu.SemaphoreType.DMA,
            pltpu.SemaphoreType.DMA,
            pltpu.SemaphoreType.DMA,
        ],
    )
    def _agg(idx_hbm, h_hbm, zeros_hbm, out_hbm,
             ibufa, ibufb, msga, msgb, acc,
             gsema, gsemb, ssema, ssemb):
        cid = lax.axis_index("c")
        sid = lax.axis_index("s")
        wid = sid * NC + cid

        def _zero(k, carry):
            c = sid + k * NS

            @pl.when(c < NZC)
            def _():
                pltpu.sync_copy(zeros_hbm, acc.at[pl.ds(c * ZR, ZR)])

            return carry

        lax.fori_loop(0, (NZC + NS - 1) // NS, _zero, None)
        plsc.subcore_barrier()

        sets = ((ibufa, msga, gsema, ssema),
                (ibufb, msgb, gsemb, ssemb))

        def _body(k, carry):
            for half, (ibuf, msg, gsem, ssem) in enumerate(sets):
                b = wid + (2 * k + half) * NW

                @pl.when(b < NBLK)
                def _():
                    @pl.when(k > 0)
                    def _():
                        for t in range(SB):
                            pltpu.make_async_copy(
                                msg.at[t], acc.at[ibuf.at[t, 1]], ssem).wait()

                    pltpu.sync_copy(idx_hbm.at[pl.ds(b * SB, SB)], ibuf)
                    for t in range(SB):
                        pltpu.async_copy(h_hbm.at[ibuf.at[t, 0]], msg.at[t],
                                         gsem)

            for half, (ibuf, msg, gsem, ssem) in enumerate(sets):
                b = wid + (2 * k + half) * NW

                @pl.when(b < NBLK)
                def _():
                    for t in range(SB):
                        pltpu.make_async_copy(
                            h_hbm.at[ibuf.at[t, 0]], msg.at[t], gsem).wait()
                    for t in range(SB):
                        pltpu.async_copy(msg.at[t], acc.at[ibuf.at[t, 1]],
                                         ssem, add=True)

            return carry

        lax.fori_loop(0, (NBLK + 2 * NW - 1) // (2 * NW), _body, None)
        for ibuf, msg, gsem, ssem in sets:
            for t in range(SB):
                pltpu.make_async_copy(
                    msg.at[t], acc.at[ibuf.at[t, 1]], ssem).wait()
        plsc.subcore_barrier()

        def _write(k, carry):
            c = sid + k * NS

            @pl.when(c < NZC)
            def _():
                pltpu.sync_copy(acc.at[pl.ds(c * ZR, ZR)],
                                out_hbm.at[cid, pl.ds(c * ZR, ZR)])

            return carry

        lax.fori_loop(0, (NZC + NS - 1) // NS, _write, None)

    return _agg


_agg16 = _make_agg(F_HID)
NP = N_NODES // 8


def _mm_body(xt_ref, w_ref, d_ref, o_ref):
    h = lax.dot_general(xt_ref[...], w_ref[...], (((0,), (0,)), ((), ())),
                        preferred_element_type=jnp.float32)
    n = lax.rsqrt(jnp.maximum(d_ref[...], 1.0)).reshape(-1, 1)
    o_ref[...] = h * n


def _matmul1(xt, w, dego_pad):
    k, m = xt.shape
    f = w.shape[1]
    bm = 2048
    nb = (m + bm - 1) // bm
    return pl.pallas_call(
        _mm_body,
        grid=(nb,),
        in_specs=[pl.BlockSpec((k, bm), lambda i: (0, i)),
                  pl.BlockSpec((k, f), lambda i: (0, 0)),
                  pl.BlockSpec((bm,), lambda i: (i,))],
        out_specs=pl.BlockSpec((bm, f), lambda i: (i, 0)),
        out_shape=jax.ShapeDtypeStruct((m, f), jnp.float32),
    )(xt, w, dego_pad)


def _packed_norm(d8, e):
    return jnp.dot(lax.rsqrt(jnp.maximum(d8, 1.0)), e,
                   preferred_element_type=jnp.float32)


def _mid_body(a_ref, di_ref, do_ref, e_ref, b1t_ref, bd_ref, o_ref):
    ni = _packed_norm(di_ref[...], e_ref[...])
    x1 = jnp.maximum((a_ref[0] + a_ref[1]) * ni + b1t_ref[...], 0.0)
    no = _packed_norm(do_ref[...], e_ref[...])
    o_ref[...] = jnp.dot(x1, bd_ref[...],
                         preferred_element_type=jnp.float32) * no


def _mid(agg1p, degi8, dego8, e, b1t, bd):
    bp = 512
    return pl.pallas_call(
        _mid_body,
        grid=((NP + bp - 1) // bp,),
        in_specs=[pl.BlockSpec((NC, bp, 128), lambda i: (0, i, 0)),
                  pl.BlockSpec((bp, 8), lambda i: (i, 0)),
                  pl.BlockSpec((bp, 8), lambda i: (i, 0)),
                  pl.BlockSpec((8, 128), lambda i: (0, 0)),
                  pl.BlockSpec((128,), lambda i: (0,)),
                  pl.BlockSpec((128, 128), lambda i: (0, 0))],
        out_specs=pl.BlockSpec((bp, 128), lambda i: (i, 0)),
        out_shape=jax.ShapeDtypeStruct((NP, 128), jnp.float32),
    )(agg1p, degi8, dego8, e, b1t, bd)


def _out_body(a_ref, di_ref, e_ref, b2t_ref, o_ref):
    ni = _packed_norm(di_ref[...], e_ref[...])
    o_ref[...] = (a_ref[0] + a_ref[1]) * ni + b2t_ref[...]


def _outk(agg2p, degi8, e, b2t):
    bp = 512
    return pl.pallas_call(
        _out_body,
        grid=((NP + bp - 1) // bp,),
        in_specs=[pl.BlockSpec((NC, bp, 128), lambda i: (0, i, 0)),
                  pl.BlockSpec((bp, 8), lambda i: (i, 0)),
                  pl.BlockSpec((8, 128), lambda i: (0, 0)),
                  pl.BlockSpec((128,), lambda i: (0,))],
        out_specs=pl.BlockSpec((bp, 128), lambda i: (i, 0)),
        out_shape=jax.ShapeDtypeStruct((NP, 128), jnp.float32),
    )(agg2p, degi8, e, b2t)


def kernel(features, edge_index, W1, b1, W2, b2):
    idx = edge_index.reshape(2, NROWS, ROW).transpose(1, 0, 2)
    dego, degi = _deg_kernel(idx, jnp.zeros((ZD,), jnp.float32))
    zeros16 = jnp.zeros((ZR, F_HID), jnp.float32)
    h1s = _matmul1(features.T, W1, dego)
    agg1 = _agg16(idx, h1s, zeros16)
    degi8 = degi.reshape(N_PAD // 8, 8)
    dego8 = dego.reshape(N_PAD // 8, 8)
    e = jnp.kron(jnp.eye(8, dtype=jnp.float32),
                 jnp.ones((1, 16), jnp.float32))
    w2p = jnp.zeros((F_HID, 16), jnp.float32).at[:, :F_OUT].set(W2)
    bd = jnp.kron(jnp.eye(8, dtype=jnp.float32), w2p)
    b1t = jnp.tile(b1, 8)
    b2t = jnp.tile(jnp.zeros((16,), jnp.float32).at[:F_OUT].set(b2), 8)
    h2p = _mid(agg1.reshape(NC, NP, 128), degi8, dego8, e, b1t, bd)
    agg2 = _agg16(idx, h2p.reshape(N_NODES, F_HID), zeros16)
    outp = _outk(agg2.reshape(NC, NP, 128), degi8, e, b2t)
    return outp.reshape(N_NODES, F_HID)[:, :F_OUT]
